# baseline (speedup 1.0000x reference)
import functools

import jax
import jax.numpy as jnp
from jax.experimental import pallas as pl
from jax.experimental.pallas import tpu as pltpu


_HI = jax.lax.Precision.HIGHEST
_BA = 4
_DN01 = (((0,), (1,)), ((), ()))
_DNT = (((1,), (1,)), ((), ()))


def _pass1_body(n_ap, n_ue, d,
                e1t_ref, e2t_ref, apt_ref, uet_ref, w1t_ref, b1_ref, w2t_ref,
                b2_ref, w3t_ref, b3_ref,
                partt_ref, ueaddt_ref, uesum_s, appre_s, uepre_s, wg_s,
                wg8_s, w3m8_s):
    j = pl.program_id(0)

    @pl.when(j == 0)
    def _init():
        wg_s[...] = w3t_ref[:, d:].astype(jnp.bfloat16)
        wg8_s[...] = jnp.concatenate(
            [w1t_ref[:, d:], w2t_ref[:, d:]],
            axis=0).astype(jnp.float8_e4m3fn)
        w3m8_s[...] = jnp.concatenate(
            [w3t_ref[:, :d], -w3t_ref[:, d:]],
            axis=1).astype(jnp.float8_e4m3fn)
        appre_s[...] = (jax.lax.dot_general(
            apt_ref[...], w1t_ref[:, :d], _DN01, precision=_HI,
            preferred_element_type=jnp.float32) + b1_ref[...])
        uepre_s[...] = (jax.lax.dot_general(
            uet_ref[...], w2t_ref[:, :d], _DN01, precision=_HI,
            preferred_element_type=jnp.float32)
            + b2_ref[...]).astype(jnp.bfloat16)
        uesum_s[...] = jnp.zeros_like(uesum_s)

    w1et = wg8_s[:d, :]
    w2et = wg8_s[d:, :]
    w3gt = wg_s[...]

    f8 = jnp.float8_e4m3fn
    e1s = e1t_ref[...].astype(jnp.bfloat16).T.astype(f8)
    e2s = e2t_ref[...].astype(jnp.bfloat16).T.astype(f8)
    t1 = jax.lax.dot_general(e1s, w1et, _DNT,
                             preferred_element_type=jnp.float32
                             ).astype(jnp.bfloat16).reshape(_BA, n_ue, d)
    t2 = jax.lax.dot_general(e2s, w2et, _DNT,
                             preferred_element_type=jnp.float32
                             ).astype(jnp.bfloat16).reshape(_BA, n_ue, d)
    appre = appre_s[pl.ds(_BA * j, _BA), :].astype(jnp.bfloat16)
    r1 = jax.nn.relu(t1 + appre[:, None, :])
    r2 = jax.nn.relu(t2 + uepre_s[...][None, :, :])
    s = (r1 + r2).reshape(_BA * n_ue, d)
    uesum_s[...] += r2

    apsum = jnp.sum(r1, axis=1, dtype=jnp.float32)
    apg = jax.lax.dot_general(apsum.astype(jnp.bfloat16), w3gt, _DNT,
                              preferred_element_type=jnp.float32)
    es_cat = jnp.concatenate([e2s, s.astype(f8)], axis=1)
    p0 = jax.lax.dot_general(es_cat, w3m8_s[...], _DNT,
                             preferred_element_type=jnp.float32
                             ).astype(jnp.bfloat16).reshape(_BA, n_ue, d)
    part = p0 + apg.astype(jnp.bfloat16)[:, None, :]
    partt_ref[...] = part.reshape(_BA * n_ue, d).T

    @pl.when(j == n_ap // _BA - 1)
    def _finish():
        ue_add = (jax.lax.dot_general(
            jnp.sum(uesum_s[...].astype(jnp.float32), axis=0), w3t_ref[:, d:],
            _DNT,
            precision=_HI,
            preferred_element_type=jnp.float32) + b3_ref[...])
        ueat = ue_add.T
        ueaddt_ref[...] = jnp.concatenate([ueat] * 4, axis=1)


def _pass2_body(partt_ref, ueaddt_ref, outt_ref):
    outt_ref[...] = partt_ref[...].astype(jnp.float32) + ueaddt_ref[...]


def kernel(ap_hid, ue_hid, ue2ap_hid, ap2ue_hid, W1, b1, W2, b2, W3, b3):
    n_ap, d = ap_hid.shape
    n_ue = ue_hid.shape[0]
    E = n_ap * n_ue
    e1t = ue2ap_hid.T
    e2t = ap2ue_hid.T
    apt = ap_hid.T
    uet = ue_hid.T
    w1t = W1.T
    w2t = W2.T
    w3t = W3.T
    b1r = b1.reshape(1, d)
    b2r = b2.reshape(1, d)
    b3r = b3.reshape(1, d)

    full = lambda shape: pl.BlockSpec(shape, lambda j: (0,) * len(shape))
    ablk = pl.BlockSpec((d, _BA * n_ue), lambda j: (0, j))

    partt, ueaddt2 = pl.pallas_call(
        functools.partial(_pass1_body, n_ap, n_ue, d),
        grid=(n_ap // _BA,),
        in_specs=[
            ablk,
            ablk,
            full((d, n_ap)),
            full((d, n_ue)),
            full((d, 2 * d)),
            full((1, d)),
            full((d, 2 * d)),
            full((1, d)),
            full((d, 2 * d)),
            full((1, d)),
        ],
        out_specs=[
            ablk,
            full((d, 4 * n_ue)),
        ],
        out_shape=[
            jax.ShapeDtypeStruct((d, E), jnp.bfloat16),
            jax.ShapeDtypeStruct((d, 4 * n_ue), jnp.float32),
        ],
        scratch_shapes=[
            pltpu.VMEM((_BA, n_ue, d), jnp.bfloat16),
            pltpu.VMEM((n_ap, d), jnp.float32),
            pltpu.VMEM((n_ue, d), jnp.bfloat16),
            pltpu.VMEM((d, d), jnp.bfloat16),
            pltpu.VMEM((2 * d, d), jnp.float8_e4m3fn),
            pltpu.VMEM((d, 2 * d), jnp.float8_e4m3fn),
        ],
    )(e1t, e2t, apt, uet, w1t, b1r, w2t, b2r, w3t, b3r)

    a2blk = pl.BlockSpec((d, 4 * n_ue), lambda j: (0, j))
    outt = pl.pallas_call(
        _pass2_body,
        grid=(n_ap // 4,),
        in_specs=[a2blk, full((d, 4 * n_ue))],
        out_specs=a2blk,
        out_shape=jax.ShapeDtypeStruct((d, E), jnp.float32),
    )(partt, ueaddt2)

    return outt.T
